# baseline (speedup 1.0000x reference)
import jax
import jax.numpy as jnp
from jax import lax
from jax.experimental import pallas as pl
from jax.experimental.pallas import tpu as pltpu
from jax.experimental.pallas import tpu_sc as plsc

B, S, D = 32, 2048, 1024
L = 16
G = 64
NG = S // G


def _take(x, idx):
    return x.at[idx].get(mode="promise_in_bounds")


def _lane_sum(x, iota):
    for sh in (1, 2, 4, 8):
        x = x + _take(x, jnp.bitwise_xor(iota, sh))
    return x


def _lane_max(x, iota):
    for sh in (1, 2, 4, 8):
        x = jnp.maximum(x, _take(x, jnp.bitwise_xor(iota, sh)))
    return x


def _prefix(x, iota):
    for sh in (1, 2, 4, 8):
        shifted = _take(x, jnp.maximum(iota - sh, 0))
        x = x + jnp.where(iota >= sh, shifted, 0)
    return x


def _body(emb_hbm, mask_hbm, out_hbm,
          mask_v, cnt_v, gcnt_v, counts_v, gcnts_v, grp_v,
          idx2_v, rows_v, out_v, counts_sh, gcnts_sh, sem, sem2):
    c = lax.axis_index("c")
    s = lax.axis_index("s")
    b = c * 16 + s
    iota = lax.broadcasted_iota(jnp.int32, (L,), 0)
    zeros16 = jnp.zeros((L,), jnp.int32)
    one = jnp.int32(1)
    zero = jnp.int32(0)

    cp0 = pltpu.async_copy(mask_hbm.at[s], mask_v.at[0], sem)
    cp1 = pltpu.async_copy(mask_hbm.at[s + 16], mask_v.at[1], sem2)
    cp0.wait()
    cp1.wait()
    pubs = []
    for j in range(2):
        glo = zeros16
        ghi = zeros16
        for g4 in range(NG // 4):
            packed = zeros16
            for v in range(4):
                msum = zeros16
                for u in range(4):
                    chunk = mask_v[j, pl.ds(g4 * (4 * G) + v * G + u * L, L)]
                    msum = msum + jnp.where(chunk != 0, one, zero)
                packed = packed + (msum << (7 * v))
            tots = _lane_sum(packed, iota)
            for v in range(4):
                tot = (tots >> (7 * v)) & 127
                g = g4 * 4 + v
                if g < 16:
                    glo = jnp.where(iota == g, tot, glo)
                else:
                    ghi = jnp.where(iota == g - 16, tot, ghi)
        gcnt_v[j, pl.ds(0, L)] = glo
        gcnt_v[j, pl.ds(L, L)] = ghi
        cnt_v[j, pl.ds(0, L)] = _lane_sum(glo + ghi, iota)
        pubs.append(pltpu.async_copy(
            cnt_v.at[j], counts_sh.at[s + 16 * j], sem))
        pubs.append(pltpu.async_copy(
            gcnt_v.at[j],
            gcnts_sh.at[pl.ds((s + 16 * j) * 2 * L, 2 * L)], sem2))
    for cp in pubs:
        cp.wait()
    plsc.subcore_barrier()

    cpa = pltpu.async_copy(counts_sh, counts_v, sem)
    cpb = pltpu.async_copy(gcnts_sh, gcnts_v, sem2)
    cpa.wait()
    cpb.wait()
    t0v = jnp.full((L,), 2 * b, jnp.int32)
    t1v = jnp.full((L,), 2 * b + 1, jnp.int32)
    cumv = zeros16
    r0v = zeros16
    r1v = zeros16
    q0v = zeros16
    q1v = zeros16
    for rp in range(B):
        row = counts_v[rp, pl.ds(0, L)]
        r0v = r0v + jnp.where(cumv <= t0v, one, zero)
        r1v = r1v + jnp.where(cumv <= t1v, one, zero)
        q0v = jnp.where(cumv <= t0v, cumv, q0v)
        q1v = jnp.where(cumv <= t1v, cumv, q1v)
        cumv = cumv + row
    r0 = r0v[0] - 1
    r1 = r1v[0] - 1
    q0 = 2 * b - q0v[0]
    q1 = 2 * b + 1 - q1v[0]

    tgt = []
    for j, (rj, q, sm) in enumerate(((r0, q0, sem), (r1, q1, sem2))):
        glo = gcnts_v[pl.ds(rj * (2 * L), L)]
        ghi = gcnts_v[pl.ds(rj * (2 * L) + L, L)]
        plo = _prefix(glo, iota)
        phi = _prefix(ghi, iota) + _take(plo, jnp.full((L,), 15, jnp.int32))
        ng = (jnp.where(plo <= q, one, zero)
              + jnp.where(phi <= q, one, zero))
        gt = _lane_sum(ng, iota)[0]
        gts = jnp.minimum(gt, NG - 1)
        base = jnp.maximum(
            _lane_max(jnp.where(plo <= q, plo, zeros16), iota),
            _lane_max(jnp.where(phi <= q, phi, zeros16), iota))[0]
        qin = jnp.where(gt < NG, q - base, jnp.int32(-1))
        cp = pltpu.async_copy(
            mask_hbm.at[rj, pl.ds(gts * G, G)], grp_v.at[j], sm)
        tgt.append((gts, qin, cp))

    pos = [None, None]
    for j, (gts, qin, cp) in enumerate(tgt):
        cp.wait()
        run = zero
        posacc = zeros16
        for u in range(4):
            chunk = grp_v[j, pl.ds(u * L, L)]
            m = chunk != 0
            mi = jnp.where(m, one, zero)
            incl = _prefix(mi, iota)
            hit = m & (run + incl - 1 == qin)
            posacc = posacc + jnp.where(hit, gts * G + u * L + iota, zeros16)
            run = run + incl[15]
        pos[j] = _lane_sum(posacc, iota)

    idx2_v[...] = b * S + jnp.where(iota == 0, pos[0], pos[1])
    pltpu.async_copy(emb_hbm.at[idx2_v.at[pl.ds(0, 2)]], rows_v, sem).wait()
    for k in range(D // L):
        sl = pl.ds(k * L, L)
        out_v[sl] = (rows_v[0, sl] + rows_v[1, sl]) * 0.5
    pltpu.sync_copy(out_v, out_hbm.at[b])


def kernel(sequence_embeddings, special_tokens_mask):
    emb_flat = sequence_embeddings.reshape(B * S, D)
    mesh = plsc.VectorSubcoreMesh(core_axis_name="c", subcore_axis_name="s")
    run = pl.kernel(
        _body,
        out_type=jax.ShapeDtypeStruct((B, D), jnp.float32),
        mesh=mesh,
        scratch_types=[
            pltpu.VMEM((2, S), jnp.int32),
            pltpu.VMEM((2, L), jnp.int32),
            pltpu.VMEM((2, 2 * L), jnp.int32),
            pltpu.VMEM((B, L), jnp.int32),
            pltpu.VMEM((B * 2 * L,), jnp.int32),
            pltpu.VMEM((2, G), jnp.int32),
            pltpu.VMEM((L,), jnp.int32),
            pltpu.VMEM((2, D), jnp.float32),
            pltpu.VMEM((D,), jnp.float32),
            pltpu.VMEM_SHARED((B, L), jnp.int32),
            pltpu.VMEM_SHARED((B * 2 * L,), jnp.int32),
            pltpu.SemaphoreType.DMA,
            pltpu.SemaphoreType.DMA,
        ],
    )
    return run(emb_flat, special_tokens_mask)


# speedup vs baseline: 1.0177x; 1.0177x over previous
import jax
import jax.numpy as jnp
from jax import lax
from jax.experimental import pallas as pl
from jax.experimental.pallas import tpu as pltpu
from jax.experimental.pallas import tpu_sc as plsc

B, S, D = 32, 2048, 1024
L = 16
G = 64
NG = S // G


def _take(x, idx):
    return x.at[idx].get(mode="promise_in_bounds")


def _lane_sum(x, iota):
    for sh in (1, 2, 4, 8):
        x = x + _take(x, jnp.bitwise_xor(iota, sh))
    return x


def _lane_max(x, iota):
    for sh in (1, 2, 4, 8):
        x = jnp.maximum(x, _take(x, jnp.bitwise_xor(iota, sh)))
    return x


def _prefix(x, iota):
    for sh in (1, 2, 4, 8):
        shifted = _take(x, jnp.maximum(iota - sh, 0))
        x = x + jnp.where(iota >= sh, shifted, 0)
    return x


def _body(emb_hbm, mask_hbm, out_hbm,
          mask_v, cnt_v, gcnt_v, counts_v, gcnts_v, grp_v,
          idx2_v, rows_v, out_v, counts_sh, gcnts_sh, sem, sem2):
    c = lax.axis_index("c")
    s = lax.axis_index("s")
    b = c * 16 + s
    iota = lax.broadcasted_iota(jnp.int32, (L,), 0)
    zeros16 = jnp.zeros((L,), jnp.int32)
    one = jnp.int32(1)
    zero = jnp.int32(0)

    cp0 = pltpu.async_copy(mask_hbm.at[s], mask_v.at[0], sem)
    cp1 = pltpu.async_copy(mask_hbm.at[s + 16], mask_v.at[1], sem2)
    pubs = []
    for j in range(2):
        (cp0 if j == 0 else cp1).wait()
        glo = zeros16
        ghi = zeros16
        for g4 in range(NG // 4):
            packed = zeros16
            for v in range(4):
                msum = zeros16
                for u in range(4):
                    chunk = mask_v[j, pl.ds(g4 * (4 * G) + v * G + u * L, L)]
                    msum = msum + jnp.where(chunk != 0, one, zero)
                packed = packed + (msum << (7 * v))
            tots = _lane_sum(packed, iota)
            for v in range(4):
                tot = (tots >> (7 * v)) & 127
                g = g4 * 4 + v
                if g < 16:
                    glo = jnp.where(iota == g, tot, glo)
                else:
                    ghi = jnp.where(iota == g - 16, tot, ghi)
        gcnt_v[j, pl.ds(0, L)] = glo
        gcnt_v[j, pl.ds(L, L)] = ghi
        cnt_v[j, pl.ds(0, L)] = _lane_sum(glo + ghi, iota)
        pubs.append(pltpu.async_copy(
            cnt_v.at[j], counts_sh.at[s + 16 * j], sem))
        pubs.append(pltpu.async_copy(
            gcnt_v.at[j],
            gcnts_sh.at[pl.ds((s + 16 * j) * 2 * L, 2 * L)], sem2))
    for cp in pubs:
        cp.wait()
    plsc.subcore_barrier()

    cpa = pltpu.async_copy(counts_sh, counts_v, sem)
    cpb = pltpu.async_copy(gcnts_sh, gcnts_v, sem2)
    cpa.wait()
    t0v = jnp.full((L,), 2 * b, jnp.int32)
    t1v = jnp.full((L,), 2 * b + 1, jnp.int32)
    cumv = zeros16
    r0v = zeros16
    r1v = zeros16
    q0v = zeros16
    q1v = zeros16
    for rp in range(B):
        row = counts_v[rp, pl.ds(0, L)]
        r0v = r0v + jnp.where(cumv <= t0v, one, zero)
        r1v = r1v + jnp.where(cumv <= t1v, one, zero)
        q0v = jnp.where(cumv <= t0v, cumv, q0v)
        q1v = jnp.where(cumv <= t1v, cumv, q1v)
        cumv = cumv + row
    r0 = r0v[0] - 1
    r1 = r1v[0] - 1
    q0 = 2 * b - q0v[0]
    q1 = 2 * b + 1 - q1v[0]
    cpb.wait()

    tgt = []
    for j, (rj, q, sm) in enumerate(((r0, q0, sem), (r1, q1, sem2))):
        glo = gcnts_v[pl.ds(rj * (2 * L), L)]
        ghi = gcnts_v[pl.ds(rj * (2 * L) + L, L)]
        plo = _prefix(glo, iota)
        phi = _prefix(ghi, iota) + _take(plo, jnp.full((L,), 15, jnp.int32))
        ng = (jnp.where(plo <= q, one, zero)
              + jnp.where(phi <= q, one, zero))
        gt = _lane_sum(ng, iota)[0]
        gts = jnp.minimum(gt, NG - 1)
        base = jnp.maximum(
            _lane_max(jnp.where(plo <= q, plo, zeros16), iota),
            _lane_max(jnp.where(phi <= q, phi, zeros16), iota))[0]
        qin = jnp.where(gt < NG, q - base, jnp.int32(-1))
        cp = pltpu.async_copy(
            mask_hbm.at[rj, pl.ds(gts * G, G)], grp_v.at[j], sm)
        tgt.append((gts, qin, cp))

    pos = [None, None]
    for j, (gts, qin, cp) in enumerate(tgt):
        cp.wait()
        run = zero
        posacc = zeros16
        for u in range(4):
            chunk = grp_v[j, pl.ds(u * L, L)]
            m = chunk != 0
            mi = jnp.where(m, one, zero)
            incl = _prefix(mi, iota)
            hit = m & (run + incl - 1 == qin)
            posacc = posacc + jnp.where(hit, gts * G + u * L + iota, zeros16)
            run = run + incl[15]
        pos[j] = _lane_sum(posacc, iota)

    idx2_v[...] = b * S + jnp.where(iota == 0, pos[0], pos[1])
    pltpu.async_copy(emb_hbm.at[idx2_v.at[pl.ds(0, 2)]], rows_v, sem).wait()
    for k in range(D // L):
        sl = pl.ds(k * L, L)
        out_v[sl] = (rows_v[0, sl] + rows_v[1, sl]) * 0.5
    pltpu.sync_copy(out_v, out_hbm.at[b])


def kernel(sequence_embeddings, special_tokens_mask):
    emb_flat = sequence_embeddings.reshape(B * S, D)
    mesh = plsc.VectorSubcoreMesh(core_axis_name="c", subcore_axis_name="s")
    run = pl.kernel(
        _body,
        out_type=jax.ShapeDtypeStruct((B, D), jnp.float32),
        mesh=mesh,
        scratch_types=[
            pltpu.VMEM((2, S), jnp.int32),
            pltpu.VMEM((2, L), jnp.int32),
            pltpu.VMEM((2, 2 * L), jnp.int32),
            pltpu.VMEM((B, L), jnp.int32),
            pltpu.VMEM((B * 2 * L,), jnp.int32),
            pltpu.VMEM((2, G), jnp.int32),
            pltpu.VMEM((L,), jnp.int32),
            pltpu.VMEM((2, D), jnp.float32),
            pltpu.VMEM((D,), jnp.float32),
            pltpu.VMEM_SHARED((B, L), jnp.int32),
            pltpu.VMEM_SHARED((B * 2 * L,), jnp.int32),
            pltpu.SemaphoreType.DMA,
            pltpu.SemaphoreType.DMA,
        ],
    )
    return run(emb_flat, special_tokens_mask)


# speedup vs baseline: 1.0414x; 1.0232x over previous
import jax
import jax.numpy as jnp
from jax import lax
from jax.experimental import pallas as pl
from jax.experimental.pallas import tpu as pltpu
from jax.experimental.pallas import tpu_sc as plsc

B, S, D = 32, 2048, 1024
L = 16
G = 64
NG = S // G


def _take(x, idx):
    return x.at[idx].get(mode="promise_in_bounds")


def _lane_sum(x, iota):
    for sh in (1, 2, 4, 8):
        x = x + _take(x, jnp.bitwise_xor(iota, sh))
    return x


def _lane_max(x, iota):
    for sh in (1, 2, 4, 8):
        x = jnp.maximum(x, _take(x, jnp.bitwise_xor(iota, sh)))
    return x


def _prefix(x, iota):
    for sh in (1, 2, 4, 8):
        shifted = _take(x, jnp.maximum(iota - sh, 0))
        x = x + jnp.where(iota >= sh, shifted, 0)
    return x


def _body(emb_hbm, mask_hbm, out_hbm,
          mask_v, cnt_v, gcnt_v, counts_v, gcnts_v, grp_v,
          idx2_v, rows_v, out_v, counts_sh, gcnts_sh, mask_sh, sem, sem2):
    c = lax.axis_index("c")
    s = lax.axis_index("s")
    b = c * 16 + s
    iota = lax.broadcasted_iota(jnp.int32, (L,), 0)
    zeros16 = jnp.zeros((L,), jnp.int32)
    one = jnp.int32(1)
    zero = jnp.int32(0)

    cp0 = pltpu.async_copy(mask_hbm.at[s], mask_v.at[0], sem)
    cp1 = pltpu.async_copy(mask_hbm.at[s + 16], mask_v.at[1], sem2)
    pubs = []
    for j in range(2):
        (cp0 if j == 0 else cp1).wait()
        pubs.append(pltpu.async_copy(
            mask_v.at[j], mask_sh.at[s + 16 * j], sem))
        glo = zeros16
        ghi = zeros16
        for g4 in range(NG // 4):
            packed = zeros16
            for v in range(4):
                msum = zeros16
                for u in range(4):
                    chunk = mask_v[j, pl.ds(g4 * (4 * G) + v * G + u * L, L)]
                    msum = msum + jnp.where(chunk != 0, one, zero)
                packed = packed + (msum << (7 * v))
            tots = _lane_sum(packed, iota)
            for v in range(4):
                tot = (tots >> (7 * v)) & 127
                g = g4 * 4 + v
                if g < 16:
                    glo = jnp.where(iota == g, tot, glo)
                else:
                    ghi = jnp.where(iota == g - 16, tot, ghi)
        gcnt_v[j, pl.ds(0, L)] = glo
        gcnt_v[j, pl.ds(L, L)] = ghi
        cnt_v[j, pl.ds(0, L)] = _lane_sum(glo + ghi, iota)
        pubs.append(pltpu.async_copy(
            cnt_v.at[j], counts_sh.at[s + 16 * j], sem))
        pubs.append(pltpu.async_copy(
            gcnt_v.at[j],
            gcnts_sh.at[pl.ds((s + 16 * j) * 2 * L, 2 * L)], sem2))
    for cp in pubs:
        cp.wait()
    plsc.subcore_barrier()

    cpa = pltpu.async_copy(counts_sh, counts_v, sem)
    cpb = pltpu.async_copy(gcnts_sh, gcnts_v, sem2)
    cpa.wait()
    t0v = jnp.full((L,), 2 * b, jnp.int32)
    t1v = jnp.full((L,), 2 * b + 1, jnp.int32)
    cumv = zeros16
    r0v = zeros16
    r1v = zeros16
    q0v = zeros16
    q1v = zeros16
    for rp in range(B):
        row = counts_v[rp, pl.ds(0, L)]
        r0v = r0v + jnp.where(cumv <= t0v, one, zero)
        r1v = r1v + jnp.where(cumv <= t1v, one, zero)
        q0v = jnp.where(cumv <= t0v, cumv, q0v)
        q1v = jnp.where(cumv <= t1v, cumv, q1v)
        cumv = cumv + row
    r0 = r0v[0] - 1
    r1 = r1v[0] - 1
    q0 = 2 * b - q0v[0]
    q1 = 2 * b + 1 - q1v[0]
    cpb.wait()

    tgt = []
    for j, (rj, q, sm) in enumerate(((r0, q0, sem), (r1, q1, sem2))):
        glo = gcnts_v[pl.ds(rj * (2 * L), L)]
        ghi = gcnts_v[pl.ds(rj * (2 * L) + L, L)]
        plo = _prefix(glo, iota)
        phi = _prefix(ghi, iota) + _take(plo, jnp.full((L,), 15, jnp.int32))
        ng = (jnp.where(plo <= q, one, zero)
              + jnp.where(phi <= q, one, zero))
        gt = _lane_sum(ng, iota)[0]
        gts = jnp.minimum(gt, NG - 1)
        base = jnp.maximum(
            _lane_max(jnp.where(plo <= q, plo, zeros16), iota),
            _lane_max(jnp.where(phi <= q, phi, zeros16), iota))[0]
        qin = jnp.where(gt < NG, q - base, jnp.int32(-1))
        cp = pltpu.async_copy(
            mask_sh.at[rj, pl.ds(gts * G, G)], grp_v.at[j], sm)
        tgt.append((gts, qin, cp))

    pos = [None, None]
    for j, (gts, qin, cp) in enumerate(tgt):
        cp.wait()
        run = zero
        posacc = zeros16
        for u in range(4):
            chunk = grp_v[j, pl.ds(u * L, L)]
            m = chunk != 0
            mi = jnp.where(m, one, zero)
            incl = _prefix(mi, iota)
            hit = m & (run + incl - 1 == qin)
            posacc = posacc + jnp.where(hit, gts * G + u * L + iota, zeros16)
            run = run + incl[15]
        pos[j] = _lane_sum(posacc, iota)

    idx2_v[...] = b * S + jnp.where(iota == 0, pos[0], pos[1])
    pltpu.async_copy(emb_hbm.at[idx2_v.at[pl.ds(0, 2)]], rows_v, sem).wait()
    for k in range(D // L):
        sl = pl.ds(k * L, L)
        out_v[sl] = (rows_v[0, sl] + rows_v[1, sl]) * 0.5
    pltpu.sync_copy(out_v, out_hbm.at[b])


def kernel(sequence_embeddings, special_tokens_mask):
    emb_flat = sequence_embeddings.reshape(B * S, D)
    mesh = plsc.VectorSubcoreMesh(core_axis_name="c", subcore_axis_name="s")
    run = pl.kernel(
        _body,
        out_type=jax.ShapeDtypeStruct((B, D), jnp.float32),
        mesh=mesh,
        scratch_types=[
            pltpu.VMEM((2, S), jnp.int32),
            pltpu.VMEM((2, L), jnp.int32),
            pltpu.VMEM((2, 2 * L), jnp.int32),
            pltpu.VMEM((B, L), jnp.int32),
            pltpu.VMEM((B * 2 * L,), jnp.int32),
            pltpu.VMEM((2, G), jnp.int32),
            pltpu.VMEM((L,), jnp.int32),
            pltpu.VMEM((2, D), jnp.float32),
            pltpu.VMEM((D,), jnp.float32),
            pltpu.VMEM_SHARED((B, L), jnp.int32),
            pltpu.VMEM_SHARED((B * 2 * L,), jnp.int32),
            pltpu.VMEM_SHARED((B, S), jnp.int32),
            pltpu.SemaphoreType.DMA,
            pltpu.SemaphoreType.DMA,
        ],
    )
    return run(emb_flat, special_tokens_mask)
